# baseline (speedup 1.0000x reference)
import jax
import jax.numpy as jnp
from jax import lax
from jax.experimental import pallas as pl
from jax.experimental.pallas import tpu as pltpu
from jax.experimental.pallas import tpu_sc as plsc

N = 10000
E = 320000
D = 128
EPS = 1e-5

NT = 32
EPT = E // NT
C = 64
G = C // 16
BLK = 256
CPB = BLK // C
MAIN = EPT // C
PAIRS = MAIN // 2
TAIL = EPT - MAIN * C
EPAD = E + BLK
RPT = 624
ZR = 16


def _dense_body(feat_ref, gamma_ref, beta_ref, wq_ref, bq_ref, wk_ref,
                wv_ref, q_ref, k_ref, v_ref):
    f = feat_ref[...]
    mean = jnp.mean(f, axis=0, keepdims=True)
    var = jnp.mean(f * f, axis=0, keepdims=True) - mean * mean
    x = (f - mean) * jax.lax.rsqrt(var + EPS) * gamma_ref[...] + beta_ref[...]
    q = jnp.dot(x, wq_ref[...], preferred_element_type=jnp.float32) \
        + bq_ref[...]
    k = jnp.dot(x, wk_ref[...], preferred_element_type=jnp.float32)
    q_ref[...] = jnp.exp(-q)
    k_ref[...] = jnp.exp(-k)
    v_ref[...] = jnp.dot(x, wv_ref[...], preferred_element_type=jnp.float32)


def _dense(feat, gamma, beta, Wq, bq, Wk, Wv):
    out = jax.ShapeDtypeStruct((N, D), jnp.float32)
    return pl.pallas_call(
        _dense_body,
        out_shape=(out, out, out),
    )(feat, gamma.reshape(1, D), beta.reshape(1, D), Wq, bq.reshape(1, D),
      Wk, Wv)


def _vgather(x, idx):
    dnums = lax.GatherDimensionNumbers(
        offset_dims=(), collapsed_slice_dims=(0,), start_index_map=(0,))
    return lax.gather(x, idx[:, None], dnums, (1,),
                      mode=lax.GatherScatterMode.PROMISE_IN_BOUNDS)


def _edge_body(q_hbm, k_hbm, v_hbm, src_hbm, dst_hbm, we_hbm,
               u_out, s_out,
               we_v, src_blk, dst_blk, dst_vt,
               q0, k0, v0, sb0, dv0,
               q1, k1, v1, sb1, dv1,
               u_acc, s_acc,
               gq0, gk0, gv0, su0, ss0,
               gq1, gk1, gv1, su1, ss1):
    cid = lax.axis_index("c")
    sid = lax.axis_index("s")
    wid = sid * 2 + cid
    ebase = wid * EPT

    pltpu.sync_copy(we_hbm, we_v)
    lanes = lax.iota(jnp.int32, 16)

    slots = ((q0, k0, v0, sb0, dv0, gq0, gk0, gv0, su0, ss0),
             (q1, k1, v1, sb1, dv1, gq1, gk1, gv1, su1, ss1))

    def zrow(i, _):
        for j in range(8):
            q0[i, pl.ds(j * 16, 16)] = jnp.zeros((16,), jnp.float32)
        return _

    lax.fori_loop(0, ZR, zrow, None)
    for j in range(4):
        sb0[pl.ds(j * 16, 16)] = jnp.zeros((16,), jnp.float32)

    row0 = sid * RPT
    ncopies = jnp.where(sid == 15, (RPT + 16) // ZR, RPT // ZR)

    def zcopy(i, _):
        pltpu.sync_copy(q0.at[pl.ds(0, ZR)],
                        u_acc.at[pl.ds(row0 + i * ZR, ZR)])
        pltpu.sync_copy(sb0.at[pl.ds(0, ZR)],
                        s_acc.at[pl.ds(row0 + i * ZR, ZR)])
        return _

    lax.fori_loop(0, ncopies, zcopy, None)
    plsc.subcore_barrier()

    def load_block(b):
        pltpu.sync_copy(src_hbm.at[pl.ds(ebase + b * BLK, BLK)], src_blk)
        pltpu.sync_copy(dst_hbm.at[pl.ds(ebase + b * BLK, BLK)], dst_blk)

    def copy_dst(dv, off):
        for j in range(4):
            dv[pl.ds(j * 16, 16)] = dst_blk[pl.ds(off + j * 16, 16)]

    def issue_gathers(t, s):
        qb, kb, vb, _, dv, gq, gk, gv, _, _ = slots[s]
        off = (t % CPB) * C
        cq = pltpu.async_copy(q_hbm.at[src_blk.at[pl.ds(off, C)]], qb, gq)
        ck = pltpu.async_copy(k_hbm.at[dv], kb, gk)
        cv = pltpu.async_copy(v_hbm.at[src_blk.at[pl.ds(off, C)]], vb, gv)
        return cq, ck, cv

    def wait_gathers(s):
        qb, kb, vb, _, _, gq, gk, gv, _, _ = slots[s]
        pltpu.make_async_copy(q_hbm.at[pl.ds(0, C)], qb, gq).wait()
        pltpu.make_async_copy(k_hbm.at[pl.ds(0, C)], kb, gk).wait()
        pltpu.make_async_copy(v_hbm.at[pl.ds(0, C)], vb, gv).wait()

    def wait_scatters(s):
        qb, kb, vb, sb, _, _, _, _, su, ss = slots[s]
        pltpu.make_async_copy(q_hbm.at[pl.ds(0, C)], vb, su).wait()
        pltpu.make_async_copy(s_out.at[0, pl.ds(0, C)], sb, ss).wait()

    def compute_p(s, ngroups):
        qb, kb, vb, sb, _, _, _, _, _, _ = slots[s]

        def group_body(m, _):
            def edge_body(l, pv):
                e = m * 16 + l
                acc = jnp.zeros((16,), jnp.float32)
                for j in range(8):
                    den = 1.0 + (qb[e, pl.ds(j * 16, 16)]
                                 * kb[e, pl.ds(j * 16, 16)])
                    acc = acc + we_v[pl.ds(j * 16, 16)] / den
                for sh in (8, 4, 2, 1):
                    acc = acc + _vgather(acc, (lanes + sh) % 16)
                return jnp.where(lanes == l, acc, pv)

            pv = lax.fori_loop(0, 16, edge_body, jnp.zeros((16,), jnp.float32))
            sb[pl.ds(m * 16, 16)] = jnp.exp(pv)
            return _

        lax.fori_loop(0, ngroups, group_body, None)

        def scale_body(m, _):
            pvec = sb[pl.ds(m * 16, 16)]

            def edge_scale(l, _):
                e = m * 16 + l
                pe = _vgather(pvec, jnp.full((16,), l, jnp.int32))
                for j in range(8):
                    vb[e, pl.ds(j * 16, 16)] = vb[e, pl.ds(j * 16, 16)] * pe
                return _

            lax.fori_loop(0, 16, edge_scale, None)
            return _

        lax.fori_loop(0, ngroups, scale_body, None)

    def issue_scatters(s):
        qb, kb, vb, sb, dv, _, _, _, su, ss = slots[s]
        pltpu.async_copy(vb, u_acc.at[dv], su, add=True)
        pltpu.async_copy(sb, s_acc.at[dv], ss, add=True)

    load_block(0)
    copy_dst(dv0, 0)
    issue_gathers(0, 0)

    def pair_body(u, _):
        t0 = 2 * u
        wait_gathers(0)

        @pl.when(u > 0)
        def _():
            wait_scatters(1)

        copy_dst(dv1, ((t0 + 1) % CPB) * C)
        issue_gathers(t0 + 1, 1)
        compute_p(0, G)
        issue_scatters(0)

        t1 = 2 * u + 1
        wait_gathers(1)

        @pl.when(lax.rem(u, 2) == 1)
        def _():
            load_block((u + 1) // 2)

        wait_scatters(0)

        @pl.when(u < PAIRS - 1)
        def _():
            copy_dst(dv0, ((t1 + 1) % CPB) * C)
            issue_gathers(t1 + 1, 0)

        compute_p(1, G)
        issue_scatters(1)
        return _

    lax.fori_loop(0, PAIRS, pair_body, None)

    wait_scatters(1)
    toff = (MAIN % CPB) * C
    for j in range(4):
        dv0[pl.ds(j * 16, 16)] = dst_blk[pl.ds(toff + j * 16, 16)]
    dst_vt[...] = dst_blk[pl.ds(toff, 16)]
    cq, ck, cv = issue_gathers(MAIN, 0)
    cq.wait()
    ck.wait()
    cv.wait()
    compute_p(0, 1)
    pltpu.sync_copy(v0.at[pl.ds(0, 16)], u_acc.at[dst_vt], add=True)
    pltpu.sync_copy(sb0.at[pl.ds(0, 16)], s_acc.at[dst_vt], add=True)

    plsc.subcore_barrier()

    def wcopy(i, _):
        r = row0 + i * ZR
        pltpu.sync_copy(u_acc.at[pl.ds(r, ZR)],
                        u_out.at[cid, pl.ds(r, ZR)])
        pltpu.sync_copy(s_acc.at[pl.ds(r, ZR)], sb0.at[pl.ds(0, ZR)])
        pltpu.sync_copy(sb0.at[pl.ds(0, ZR)], s_out.at[cid, pl.ds(r, ZR)])
        return _

    lax.fori_loop(0, ncopies, wcopy, None)


def _edge_sc(q, k, v, src, dst, we):
    mesh = plsc.VectorSubcoreMesh(core_axis_name="c", subcore_axis_name="s")
    f32 = jnp.float32
    i32 = jnp.int32
    kfn = pl.kernel(
        _edge_body,
        out_type=(jax.ShapeDtypeStruct((2, N, D), f32),
                  jax.ShapeDtypeStruct((2, N), f32)),
        mesh=mesh,
        scratch_types=[
            pltpu.VMEM((D,), f32),
            pltpu.VMEM((BLK,), i32),
            pltpu.VMEM((BLK,), i32),
            pltpu.VMEM((16,), i32),
            pltpu.VMEM((C, D), f32),
            pltpu.VMEM((C, D), f32),
            pltpu.VMEM((C, D), f32),
            pltpu.VMEM((C,), f32),
            pltpu.VMEM((C,), i32),
            pltpu.VMEM((C, D), f32),
            pltpu.VMEM((C, D), f32),
            pltpu.VMEM((C, D), f32),
            pltpu.VMEM((C,), f32),
            pltpu.VMEM((C,), i32),
            pltpu.VMEM_SHARED((N, D), f32),
            pltpu.VMEM_SHARED((N,), f32),
            pltpu.SemaphoreType.DMA,
            pltpu.SemaphoreType.DMA,
            pltpu.SemaphoreType.DMA,
            pltpu.SemaphoreType.DMA,
            pltpu.SemaphoreType.DMA,
            pltpu.SemaphoreType.DMA,
            pltpu.SemaphoreType.DMA,
            pltpu.SemaphoreType.DMA,
            pltpu.SemaphoreType.DMA,
            pltpu.SemaphoreType.DMA,
        ],
    )
    return kfn(q, k, v, src, dst, we)


def _final_body(u_ref, s_ref, o_ref):
    u = u_ref[0] + u_ref[1]
    s = (s_ref[0] + s_ref[1])[:, None]
    o_ref[...] = u / jnp.maximum(s, 1e-30)


def _finalize(U, S):
    return pl.pallas_call(
        _final_body,
        out_shape=jax.ShapeDtypeStruct((N, D), jnp.float32),
    )(U, S)


def kernel(feat, edge_index, gamma, beta, Wq, bq, Wk, Wv, We):
    q, k, v = _dense(feat, gamma, beta, Wq, bq, Wk, Wv)
    src = jnp.pad(edge_index[0], (0, EPAD - E))
    dst = jnp.pad(edge_index[1], (0, EPAD - E))
    U, S = _edge_sc(q, k, v, src, dst, We.reshape(D))
    return _finalize(U, S)


# speedup vs baseline: 1.1182x; 1.1182x over previous
import jax
import jax.numpy as jnp
from jax import lax
from jax.experimental import pallas as pl
from jax.experimental.pallas import tpu as pltpu
from jax.experimental.pallas import tpu_sc as plsc

N = 10000
E = 320000
D = 128
EPS = 1e-5

NT = 32
EPT = E // NT
C = 64
G = C // 16
BLK = 256
CPB = BLK // C
MAIN = EPT // C
PAIRS = MAIN // 2
TAIL = EPT - MAIN * C
EPAD = E + BLK
RPT = 624
ZR = 16


def _dense_body(feat_ref, gamma_ref, beta_ref, wq_ref, bq_ref, wk_ref,
                wv_ref, q_ref, k_ref, v_ref):
    f = feat_ref[...]
    mean = jnp.mean(f, axis=0, keepdims=True)
    var = jnp.mean(f * f, axis=0, keepdims=True) - mean * mean
    x = (f - mean) * jax.lax.rsqrt(var + EPS) * gamma_ref[...] + beta_ref[...]
    q = jnp.dot(x, wq_ref[...], preferred_element_type=jnp.float32) \
        + bq_ref[...]
    k = jnp.dot(x, wk_ref[...], preferred_element_type=jnp.float32)
    q_ref[...] = jnp.exp(-q)
    k_ref[...] = jnp.exp(-k)
    v_ref[...] = jnp.dot(x, wv_ref[...], preferred_element_type=jnp.float32)


def _dense(feat, gamma, beta, Wq, bq, Wk, Wv):
    out = jax.ShapeDtypeStruct((N, D), jnp.float32)
    return pl.pallas_call(
        _dense_body,
        out_shape=(out, out, out),
    )(feat, gamma.reshape(1, D), beta.reshape(1, D), Wq, bq.reshape(1, D),
      Wk, Wv)


def _vgather(x, idx):
    dnums = lax.GatherDimensionNumbers(
        offset_dims=(), collapsed_slice_dims=(0,), start_index_map=(0,))
    return lax.gather(x, idx[:, None], dnums, (1,),
                      mode=lax.GatherScatterMode.PROMISE_IN_BOUNDS)


def _edge_body(q_hbm, k_hbm, v_hbm, src_hbm, dst_hbm, we_hbm,
               u_out, s_out,
               we_v, src_blk, dst_blk, dst_vt,
               q0, k0, v0, sb0, dv0,
               q1, k1, v1, sb1, dv1,
               u_acc, s_acc,
               gq0, gk0, gv0, su0, ss0,
               gq1, gk1, gv1, su1, ss1, bk0, bk1):
    cid = lax.axis_index("c")
    sid = lax.axis_index("s")
    wid = sid * 2 + cid
    ebase = wid * EPT

    pltpu.sync_copy(we_hbm, we_v)
    lanes = lax.iota(jnp.int32, 16)

    slots = ((q0, k0, v0, sb0, dv0, gq0, gk0, gv0, su0, ss0),
             (q1, k1, v1, sb1, dv1, gq1, gk1, gv1, su1, ss1))

    def zrow(i, _):
        for j in range(8):
            q0[i, pl.ds(j * 16, 16)] = jnp.zeros((16,), jnp.float32)
        return _

    lax.fori_loop(0, ZR, zrow, None)
    for j in range(4):
        sb0[pl.ds(j * 16, 16)] = jnp.zeros((16,), jnp.float32)

    row0 = sid * RPT
    NZC = (RPT + 16) // ZR

    def zissue(i, _):
        pltpu.async_copy(q0.at[pl.ds(0, ZR)],
                         u_acc.at[pl.ds(row0 + i * ZR, ZR)], gq0)
        return _

    lax.fori_loop(0, NZC, zissue, None)
    for j in range(5):
        pltpu.async_copy(q0.at[2 * j], s_acc.at[pl.ds(row0 + j * 128, 128)], gk0)

    def zdrain(i, _):
        pltpu.make_async_copy(q0.at[pl.ds(0, ZR)],
                              u_acc.at[pl.ds(0, ZR)], gq0).wait()
        return _

    lax.fori_loop(0, NZC, zdrain, None)
    for j in range(5):
        pltpu.make_async_copy(q0.at[2 * j], s_acc.at[pl.ds(0, 128)], gk0).wait()
    plsc.subcore_barrier()

    def load_block(b):
        cs = pltpu.async_copy(src_hbm.at[pl.ds(ebase + b * BLK, BLK)],
                              src_blk, bk0)
        cd = pltpu.async_copy(dst_hbm.at[pl.ds(ebase + b * BLK, BLK)],
                              dst_blk, bk1)
        cs.wait()
        cd.wait()

    def copy_dst(dv, off):
        for j in range(4):
            dv[pl.ds(j * 16, 16)] = dst_blk[pl.ds(off + j * 16, 16)]

    def issue_gathers(t, s):
        qb, kb, vb, _, dv, gq, gk, gv, _, _ = slots[s]
        off = (t % CPB) * C
        cq = pltpu.async_copy(q_hbm.at[src_blk.at[pl.ds(off, C)]], qb, gq)
        ck = pltpu.async_copy(k_hbm.at[dv], kb, gk)
        cv = pltpu.async_copy(v_hbm.at[src_blk.at[pl.ds(off, C)]], vb, gv)
        return cq, ck, cv

    def wait_gathers(s):
        qb, kb, vb, _, _, gq, gk, gv, _, _ = slots[s]
        pltpu.make_async_copy(q_hbm.at[pl.ds(0, C)], qb, gq).wait()
        pltpu.make_async_copy(k_hbm.at[pl.ds(0, C)], kb, gk).wait()
        pltpu.make_async_copy(v_hbm.at[pl.ds(0, C)], vb, gv).wait()

    def wait_scatters(s):
        qb, kb, vb, sb, _, _, _, _, su, ss = slots[s]
        pltpu.make_async_copy(q_hbm.at[pl.ds(0, C)], vb, su).wait()
        pltpu.make_async_copy(s_out.at[pl.ds(0, C)], sb, ss).wait()

    def compute_p(s, ngroups):
        qb, kb, vb, sb, _, _, _, _, _, _ = slots[s]

        def group_body(m, _):
            def edge_body(l, pv):
                e = m * 16 + l
                acc = jnp.zeros((16,), jnp.float32)
                for j in range(8):
                    den = 1.0 + (qb[e, pl.ds(j * 16, 16)]
                                 * kb[e, pl.ds(j * 16, 16)])
                    acc = acc + we_v[pl.ds(j * 16, 16)] / den
                for sh in (8, 4, 2, 1):
                    acc = acc + _vgather(acc, (lanes + sh) % 16)
                return jnp.where(lanes == l, acc, pv)

            pv = lax.fori_loop(0, 16, edge_body, jnp.zeros((16,), jnp.float32))
            sb[pl.ds(m * 16, 16)] = jnp.exp(pv)
            return _

        lax.fori_loop(0, ngroups, group_body, None)

        def scale_body(m, _):
            pvec = sb[pl.ds(m * 16, 16)]

            def edge_scale(l, _):
                e = m * 16 + l
                pe = _vgather(pvec, jnp.full((16,), l, jnp.int32))
                for j in range(8):
                    vb[e, pl.ds(j * 16, 16)] = vb[e, pl.ds(j * 16, 16)] * pe
                return _

            lax.fori_loop(0, 16, edge_scale, None)
            return _

        lax.fori_loop(0, ngroups, scale_body, None)

    def issue_scatters(s):
        qb, kb, vb, sb, dv, _, _, _, su, ss = slots[s]
        pltpu.async_copy(vb, u_acc.at[dv], su, add=True)
        pltpu.async_copy(sb, s_acc.at[dv], ss, add=True)

    load_block(0)
    copy_dst(dv0, 0)
    issue_gathers(0, 0)

    def pair_body(u, _):
        t0 = 2 * u
        wait_gathers(0)

        @pl.when(u > 0)
        def _():
            wait_scatters(1)

        copy_dst(dv1, ((t0 + 1) % CPB) * C)
        issue_gathers(t0 + 1, 1)
        compute_p(0, G)
        issue_scatters(0)

        t1 = 2 * u + 1
        wait_gathers(1)

        @pl.when(lax.rem(u, 2) == 1)
        def _():
            load_block((u + 1) // 2)

        wait_scatters(0)

        @pl.when(u < PAIRS - 1)
        def _():
            copy_dst(dv0, ((t1 + 1) % CPB) * C)
            issue_gathers(t1 + 1, 0)

        compute_p(1, G)
        issue_scatters(1)
        return _

    lax.fori_loop(0, PAIRS, pair_body, None)

    wait_scatters(1)
    toff = (MAIN % CPB) * C
    for j in range(4):
        dv0[pl.ds(j * 16, 16)] = dst_blk[pl.ds(toff + j * 16, 16)]
    dst_vt[...] = dst_blk[pl.ds(toff, 16)]
    cq, ck, cv = issue_gathers(MAIN, 0)
    cq.wait()
    ck.wait()
    cv.wait()
    compute_p(0, 1)
    pltpu.sync_copy(v0.at[pl.ds(0, 16)], u_acc.at[dst_vt], add=True)
    pltpu.sync_copy(sb0.at[pl.ds(0, 16)], s_acc.at[dst_vt], add=True)

    plsc.subcore_barrier()

    cu = pltpu.async_copy(u_acc.at[pl.ds(row0, RPT)],
                          u_out.at[cid, pl.ds(row0, RPT)], gq0)
    for j in range(5):
        pltpu.async_copy(s_acc.at[pl.ds(row0 + j * 128, 128)], q0.at[2 * j], gk0)

    @pl.when(sid == 15)
    def _():
        pltpu.async_copy(u_acc.at[pl.ds(row0 + RPT, 16)],
                         u_out.at[cid, pl.ds(row0 + RPT, 16)], gv0)

    for j in range(5):
        pltpu.make_async_copy(s_acc.at[pl.ds(0, 128)], q0.at[2 * j], gk0).wait()
    for j in range(5):
        pltpu.async_copy(q0.at[2 * j],
                         s_out.at[pl.ds(cid * N + row0 + j * 128, 128)], gk0)
    for j in range(5):
        pltpu.make_async_copy(q0.at[2 * j], s_out.at[pl.ds(0, 128)],
                              gk0).wait()
    cu.wait()

    @pl.when(sid == 15)
    def _():
        pltpu.make_async_copy(u_acc.at[pl.ds(0, 16)],
                              u_out.at[0, pl.ds(0, 16)], gv0).wait()


def _edge_sc(q, k, v, src, dst, we):
    mesh = plsc.VectorSubcoreMesh(core_axis_name="c", subcore_axis_name="s")
    f32 = jnp.float32
    i32 = jnp.int32
    kfn = pl.kernel(
        _edge_body,
        out_type=(jax.ShapeDtypeStruct((2, N, D), f32),
                  jax.ShapeDtypeStruct((2 * N,), f32)),
        mesh=mesh,
        scratch_types=[
            pltpu.VMEM((D,), f32),
            pltpu.VMEM((BLK,), i32),
            pltpu.VMEM((BLK,), i32),
            pltpu.VMEM((16,), i32),
            pltpu.VMEM((C, D), f32),
            pltpu.VMEM((C, D), f32),
            pltpu.VMEM((C, D), f32),
            pltpu.VMEM((C,), f32),
            pltpu.VMEM((C,), i32),
            pltpu.VMEM((C, D), f32),
            pltpu.VMEM((C, D), f32),
            pltpu.VMEM((C, D), f32),
            pltpu.VMEM((C,), f32),
            pltpu.VMEM((C,), i32),
            pltpu.VMEM_SHARED((N, D), f32),
            pltpu.VMEM_SHARED((N,), f32),
            pltpu.SemaphoreType.DMA,
            pltpu.SemaphoreType.DMA,
            pltpu.SemaphoreType.DMA,
            pltpu.SemaphoreType.DMA,
            pltpu.SemaphoreType.DMA,
            pltpu.SemaphoreType.DMA,
            pltpu.SemaphoreType.DMA,
            pltpu.SemaphoreType.DMA,
            pltpu.SemaphoreType.DMA,
            pltpu.SemaphoreType.DMA,
            pltpu.SemaphoreType.DMA,
            pltpu.SemaphoreType.DMA,
        ],
    )
    return kfn(q, k, v, src, dst, we)


def _final_body(u_ref, s_ref, o_ref):
    u = u_ref[0] + u_ref[1]
    s = (s_ref[0] + s_ref[1])[:, None]
    o_ref[...] = u / jnp.maximum(s, 1e-30)


def _finalize(U, S):
    return pl.pallas_call(
        _final_body,
        out_shape=jax.ShapeDtypeStruct((N, D), jnp.float32),
    )(U, S)


def kernel(feat, edge_index, gamma, beta, Wq, bq, Wk, Wv, We):
    q, k, v = _dense(feat, gamma, beta, Wq, bq, Wk, Wv)
    src = jnp.pad(edge_index[0], (0, EPAD - E))
    dst = jnp.pad(edge_index[1], (0, EPAD - E))
    U, S = _edge_sc(q, k, v, src, dst, We.reshape(D))
    return _finalize(U, S.reshape(2, N))
